# baseline (speedup 1.0000x reference)
import functools
import math

import jax
import jax.numpy as jnp
from jax import lax
from jax.experimental import pallas as pl
from jax.experimental.pallas import tpu as pltpu
from jax.experimental.pallas import tpu_sc as plsc

N = 10000
D = 128
LAYERS = 2

L = 16
NS = 16
NC = 2
CH = 88
NR = 4
NI = 6
PER = 12
NP = 10240
RPT = NP // NS


def _make_sc_spmm(nch):
  assert nch % PER == 0 and nch // PER >= 3
  mesh = plsc.VectorSubcoreMesh(core_axis_name="c", subcore_axis_name="s")

  @functools.partial(
      pl.kernel,
      out_type=(jax.ShapeDtypeStruct((NP, D), jnp.float32),
                jax.ShapeDtypeStruct((NP, D), jnp.float32)),
      mesh=mesh,
      scratch_types=[
          pltpu.VMEM((NI, CH), jnp.int32),
          pltpu.VMEM((NI, CH), jnp.int32),
          pltpu.VMEM((NI, CH), jnp.float32),
          pltpu.VMEM((NR, CH, D), jnp.float32),
          pltpu.VMEM_SHARED((NP, D), jnp.float32),
          pltpu.SemaphoreType.DMA((NI,)),
          pltpu.SemaphoreType.DMA((NR,)),
          pltpu.SemaphoreType.DMA((NR,)),
      ],
      compiler_params=pltpu.CompilerParams(needs_layout_passes=False),
  )
  def kern(x_hbm, isrc, idst, iw, osrc, odst, ow, out_in, out_out,
           sidx, didx, wbuf, rows, acc, isem, gsem, ssem):
    cid = lax.axis_index("c")
    sid = lax.axis_index("s")
    zero16 = jnp.zeros((L,), jnp.float32)

    @pl.loop(0, CH)
    def _(e):
      for c in range(D // L):
        rows[0, e, pl.ds(c * L, L)] = zero16

    for kk in range(RPT // CH):
      pltpu.sync_copy(rows.at[0], acc.at[pl.ds(sid * RPT + kk * CH, CH)])
    rem = RPT % CH
    if rem:
      pltpu.sync_copy(
          rows.at[0].at[pl.ds(0, rem)],
          acc.at[pl.ds(sid * RPT + (RPT // CH) * CH, rem)])
    plsc.subcore_barrier()

    def run(src_h, dst_h, w_h):
      def idx_fetch(j, bi):
        base = (sid * nch + j) * CH
        pltpu.async_copy(src_h.at[pl.ds(base, CH)], sidx.at[bi], isem.at[bi])
        pltpu.async_copy(dst_h.at[pl.ds(base, CH)], didx.at[bi], isem.at[bi])
        pltpu.async_copy(w_h.at[pl.ds(base, CH)], wbuf.at[bi], isem.at[bi])

      def idx_wait(bi):
        pltpu.make_async_copy(
            src_h.at[pl.ds(0, CH)], sidx.at[bi], isem.at[bi]).wait()
        pltpu.make_async_copy(
            dst_h.at[pl.ds(0, CH)], didx.at[bi], isem.at[bi]).wait()
        pltpu.make_async_copy(
            w_h.at[pl.ds(0, CH)], wbuf.at[bi], isem.at[bi]).wait()

      def gather_start(bi, br):
        pltpu.async_copy(x_hbm.at[sidx.at[bi]], rows.at[br], gsem.at[br])

      def wait_gather(bi, br):
        pltpu.make_async_copy(
            x_hbm.at[sidx.at[bi]], rows.at[br], gsem.at[br]).wait()

      def scatter_start(bi, br):
        pltpu.async_copy(rows.at[br], acc.at[didx.at[bi]], ssem.at[br],
                         add=True)

      def wait_scatter(bi, br):
        pltpu.make_async_copy(
            rows.at[br], acc.at[didx.at[bi]], ssem.at[br]).wait()

      def multiply(bi, br):
        @pl.loop(0, CH, unroll=4)
        def _(e):
          wsplat = plsc.load_gather(
              wbuf.at[bi], [jnp.zeros((L,), jnp.int32) + e])
          for c in range(D // L):
            sl = rows[br, e, pl.ds(c * L, L)]
            rows[br, e, pl.ds(c * L, L)] = sl * wsplat

      def slot(j, kmod):
        @pl.when(j >= 2)
        def _():
          wait_scatter((kmod - 2) % NI, (kmod - 2) % NR)

        @pl.when(j + 4 < nch)
        def _():
          idx_fetch(j + 4, (kmod + 4) % NI)

        @pl.when(j + 2 < nch)
        def _():
          idx_wait((kmod + 2) % NI)
          gather_start((kmod + 2) % NI, (kmod + 2) % NR)

        wait_gather(kmod % NI, kmod % NR)
        multiply(kmod % NI, kmod % NR)
        scatter_start(kmod % NI, kmod % NR)

      for jj in range(4):
        idx_fetch(jj, jj)
      idx_wait(0)
      gather_start(0, 0)
      idx_wait(1)
      gather_start(1, 1)

      ngrp = nch // PER

      @pl.loop(0, ngrp)
      def _(g):
        j0 = g * PER
        for k in range(PER):
          slot(j0 + k, k)

      wait_scatter((PER - 2) % NI, (PER - 2) % NR)
      wait_scatter((PER - 1) % NI, (PER - 1) % NR)

    @pl.when(cid == 0)
    def _():
      run(isrc, idst, iw)

    @pl.when(cid == 1)
    def _():
      run(osrc, odst, ow)

    plsc.subcore_barrier()
    sl = pl.ds(sid * RPT, RPT)

    @pl.when(cid == 0)
    def _():
      pltpu.sync_copy(acc.at[sl], out_in.at[sl])

    @pl.when(cid == 1)
    def _():
      pltpu.sync_copy(acc.at[sl], out_out.at[sl])

  return kern


_BR = 1000
_SCALE = 1.0 / math.sqrt(D)


def _gate_body(x_ref, a_ref, b_ref, w1_ref, b1_ref, w2_ref, b2_ref, cp_ref,
               o_ref):
  x = x_ref[...]
  a = a_ref[...]
  b = b_ref[...]
  dn = (((1,), (1,)), ((), ()))
  t1 = jax.nn.relu(
      lax.dot_general(x * a, w1_ref[...], dn,
                      preferred_element_type=jnp.float32) + b1_ref[...])
  s1 = jnp.sum(t1, axis=1, keepdims=True) * _SCALE
  t2 = jax.nn.relu(
      lax.dot_general(x * b, w2_ref[...], dn,
                      preferred_element_type=jnp.float32) + b2_ref[...])
  s2 = jnp.sum(t2, axis=1, keepdims=True) * _SCALE
  m = jnp.maximum(s1, s2)
  e1 = jnp.exp(s1 - m)
  e2 = jnp.exp(s2 - m)
  inv = 1.0 / (e1 + e2)
  nb = a * (e1 * inv) + b * (e2 * inv)
  o_ref[...] = x * cp_ref[0] + nb * cp_ref[1] + cp_ref[2]


def _tc_gate(h, inb, outb, W1_w, b1, W2_w, b2, cp):
  grid = N // _BR
  blk = lambda: pl.BlockSpec((_BR, D), lambda i: (i, 0))
  full = lambda r, c: pl.BlockSpec((r, c), lambda i: (0, 0))
  return pl.pallas_call(
      _gate_body,
      grid=(grid,),
      in_specs=[
          blk(), blk(), blk(),
          full(D, D), full(1, D), full(D, D), full(1, D),
          pl.BlockSpec(memory_space=pltpu.SMEM),
      ],
      out_specs=blk(),
      out_shape=jax.ShapeDtypeStruct((N, D), jnp.float32),
  )(h, inb, outb, W1_w, b1, W2_w, b2, cp)


def kernel(x, in_edge_index, in_edge_weight, out_edge_index, out_edge_weight,
           W1_w, W1_b, W2_w, W2_b, conv_w, conv_b):
  E = in_edge_index.shape[1]
  epw = -(-E // NS)
  nch = -(-epw // CH)
  nch = max(-(-nch // PER), 3) * PER
  EP = NS * nch * CH
  pad = EP - E

  def prep(ei, ew):
    fill = jnp.arange(pad, dtype=jnp.int32) % N
    src = jnp.concatenate([ei[1], fill])
    dst = jnp.concatenate([ei[0], fill])
    w = jnp.concatenate([ew, jnp.zeros((pad,), ew.dtype)])
    return src, dst, w

  isrc, idst, iw = prep(in_edge_index, in_edge_weight)
  osrc, odst, ow = prep(out_edge_index, out_edge_weight)

  spmm = _make_sc_spmm(nch)
  b1 = W1_b.reshape(1, D)
  b2 = W2_b.reshape(1, D)
  cp = jnp.concatenate([conv_w, conv_b.reshape(1)])

  h = x
  for _ in range(LAYERS):
    inb, outb = spmm(h, isrc, idst, iw, osrc, odst, ow)
    h = _tc_gate(h, inb, outb, W1_w, b1, W2_w, b2, cp)
  return h


# speedup vs baseline: 1.0030x; 1.0030x over previous
import functools
import math

import jax
import jax.numpy as jnp
from jax import lax
from jax.experimental import pallas as pl
from jax.experimental.pallas import tpu as pltpu
from jax.experimental.pallas import tpu_sc as plsc

N = 10000
D = 128
LAYERS = 2

L = 16
NS = 16
NC = 2
CH = 88
NR = 4
NI = 6
PER = 12
NP = 10240
RPT = NP // NS


def _make_sc_spmm(nch):
  assert nch % PER == 0 and nch // PER >= 3
  mesh = plsc.VectorSubcoreMesh(core_axis_name="c", subcore_axis_name="s")

  @functools.partial(
      pl.kernel,
      out_type=(jax.ShapeDtypeStruct((NP, D), jnp.float32),
                jax.ShapeDtypeStruct((NP, D), jnp.float32)),
      mesh=mesh,
      scratch_types=[
          pltpu.VMEM((NI, CH), jnp.int32),
          pltpu.VMEM((NI, CH), jnp.int32),
          pltpu.VMEM((NI, CH), jnp.float32),
          pltpu.VMEM((NR, CH, D), jnp.float32),
          pltpu.VMEM_SHARED((NP, D), jnp.float32),
          pltpu.SemaphoreType.DMA((NI,)),
          pltpu.SemaphoreType.DMA((NR,)),
          pltpu.SemaphoreType.DMA((NR,)),
      ],
      compiler_params=pltpu.CompilerParams(needs_layout_passes=False),
  )
  def kern(x_hbm, isrc, idst, iw, osrc, odst, ow, out_in, out_out,
           sidx, didx, wbuf, rows, acc, isem, gsem, ssem):
    cid = lax.axis_index("c")
    sid = lax.axis_index("s")
    zero16 = jnp.zeros((L,), jnp.float32)

    @pl.loop(0, CH)
    def _(e):
      for c in range(D // L):
        rows[0, e, pl.ds(c * L, L)] = zero16

    for kk in range(RPT // CH):
      pltpu.sync_copy(rows.at[0], acc.at[pl.ds(sid * RPT + kk * CH, CH)])
    rem = RPT % CH
    if rem:
      pltpu.sync_copy(
          rows.at[0].at[pl.ds(0, rem)],
          acc.at[pl.ds(sid * RPT + (RPT // CH) * CH, rem)])
    plsc.subcore_barrier()

    def run(src_h, dst_h, w_h):
      def idx_fetch(j, bi):
        base = (sid * nch + j) * CH
        pltpu.async_copy(src_h.at[pl.ds(base, CH)], sidx.at[bi], isem.at[bi])
        pltpu.async_copy(dst_h.at[pl.ds(base, CH)], didx.at[bi], isem.at[bi])
        pltpu.async_copy(w_h.at[pl.ds(base, CH)], wbuf.at[bi], isem.at[bi])

      def idx_wait(bi):
        pltpu.make_async_copy(
            src_h.at[pl.ds(0, CH)], sidx.at[bi], isem.at[bi]).wait()
        pltpu.make_async_copy(
            dst_h.at[pl.ds(0, CH)], didx.at[bi], isem.at[bi]).wait()
        pltpu.make_async_copy(
            w_h.at[pl.ds(0, CH)], wbuf.at[bi], isem.at[bi]).wait()

      def gather_start(bi, br):
        pltpu.async_copy(x_hbm.at[sidx.at[bi]], rows.at[br], gsem.at[br])

      def wait_gather(bi, br):
        pltpu.make_async_copy(
            x_hbm.at[sidx.at[bi]], rows.at[br], gsem.at[br]).wait()

      def scatter_start(bi, br):
        pltpu.async_copy(rows.at[br], acc.at[didx.at[bi]], ssem.at[br],
                         add=True)

      def wait_scatter(bi, br):
        pltpu.make_async_copy(
            rows.at[br], acc.at[didx.at[bi]], ssem.at[br]).wait()

      def multiply(bi, br):
        @pl.loop(0, CH, unroll=2)
        def _(e):
          wsplat = plsc.load_gather(
              wbuf.at[bi], [jnp.zeros((L,), jnp.int32) + e])
          for c in range(D // L):
            sl = rows[br, e, pl.ds(c * L, L)]
            rows[br, e, pl.ds(c * L, L)] = sl * wsplat

      def slot(j, kmod):
        @pl.when(j >= 2)
        def _():
          wait_scatter((kmod - 2) % NI, (kmod - 2) % NR)

        @pl.when(j + 4 < nch)
        def _():
          idx_fetch(j + 4, (kmod + 4) % NI)

        @pl.when(j + 2 < nch)
        def _():
          idx_wait((kmod + 2) % NI)
          gather_start((kmod + 2) % NI, (kmod + 2) % NR)

        wait_gather(kmod % NI, kmod % NR)
        multiply(kmod % NI, kmod % NR)
        scatter_start(kmod % NI, kmod % NR)

      for jj in range(4):
        idx_fetch(jj, jj)
      idx_wait(0)
      gather_start(0, 0)
      idx_wait(1)
      gather_start(1, 1)

      ngrp = nch // PER

      @pl.loop(0, ngrp)
      def _(g):
        j0 = g * PER
        for k in range(PER):
          slot(j0 + k, k)

      wait_scatter((PER - 2) % NI, (PER - 2) % NR)
      wait_scatter((PER - 1) % NI, (PER - 1) % NR)

    @pl.when(cid == 0)
    def _():
      run(isrc, idst, iw)

    @pl.when(cid == 1)
    def _():
      run(osrc, odst, ow)

    plsc.subcore_barrier()
    sl = pl.ds(sid * RPT, RPT)

    @pl.when(cid == 0)
    def _():
      pltpu.sync_copy(acc.at[sl], out_in.at[sl])

    @pl.when(cid == 1)
    def _():
      pltpu.sync_copy(acc.at[sl], out_out.at[sl])

  return kern


_BR = 1000
_SCALE = 1.0 / math.sqrt(D)


def _gate_body(x_ref, a_ref, b_ref, w1_ref, b1_ref, w2_ref, b2_ref, cp_ref,
               o_ref):
  x = x_ref[...]
  a = a_ref[...]
  b = b_ref[...]
  dn = (((1,), (1,)), ((), ()))
  t1 = jax.nn.relu(
      lax.dot_general(x * a, w1_ref[...], dn,
                      preferred_element_type=jnp.float32) + b1_ref[...])
  s1 = jnp.sum(t1, axis=1, keepdims=True) * _SCALE
  t2 = jax.nn.relu(
      lax.dot_general(x * b, w2_ref[...], dn,
                      preferred_element_type=jnp.float32) + b2_ref[...])
  s2 = jnp.sum(t2, axis=1, keepdims=True) * _SCALE
  m = jnp.maximum(s1, s2)
  e1 = jnp.exp(s1 - m)
  e2 = jnp.exp(s2 - m)
  inv = 1.0 / (e1 + e2)
  nb = a * (e1 * inv) + b * (e2 * inv)
  o_ref[...] = x * cp_ref[0] + nb * cp_ref[1] + cp_ref[2]


def _tc_gate(h, inb, outb, W1_w, b1, W2_w, b2, cp):
  grid = N // _BR
  blk = lambda: pl.BlockSpec((_BR, D), lambda i: (i, 0))
  full = lambda r, c: pl.BlockSpec((r, c), lambda i: (0, 0))
  return pl.pallas_call(
      _gate_body,
      grid=(grid,),
      in_specs=[
          blk(), blk(), blk(),
          full(D, D), full(1, D), full(D, D), full(1, D),
          pl.BlockSpec(memory_space=pltpu.SMEM),
      ],
      out_specs=blk(),
      out_shape=jax.ShapeDtypeStruct((N, D), jnp.float32),
  )(h, inb, outb, W1_w, b1, W2_w, b2, cp)


def kernel(x, in_edge_index, in_edge_weight, out_edge_index, out_edge_weight,
           W1_w, W1_b, W2_w, W2_b, conv_w, conv_b):
  E = in_edge_index.shape[1]
  epw = -(-E // NS)
  nch = -(-epw // CH)
  nch = max(-(-nch // PER), 3) * PER
  EP = NS * nch * CH
  pad = EP - E

  def prep(ei, ew):
    fill = jnp.arange(pad, dtype=jnp.int32) % N
    src = jnp.concatenate([ei[1], fill])
    dst = jnp.concatenate([ei[0], fill])
    w = jnp.concatenate([ew, jnp.zeros((pad,), ew.dtype)])
    return src, dst, w

  isrc, idst, iw = prep(in_edge_index, in_edge_weight)
  osrc, odst, ow = prep(out_edge_index, out_edge_weight)

  spmm = _make_sc_spmm(nch)
  b1 = W1_b.reshape(1, D)
  b2 = W2_b.reshape(1, D)
  cp = jnp.concatenate([conv_w, conv_b.reshape(1)])

  h = x
  for _ in range(LAYERS):
    inb, outb = spmm(h, isrc, idst, iw, osrc, odst, ow)
    h = _tc_gate(h, inb, outb, W1_w, b1, W2_w, b2, cp)
  return h


# speedup vs baseline: 1.0042x; 1.0012x over previous
import functools
import math

import jax
import jax.numpy as jnp
from jax import lax
from jax.experimental import pallas as pl
from jax.experimental.pallas import tpu as pltpu
from jax.experimental.pallas import tpu_sc as plsc

N = 10000
D = 128
LAYERS = 2

L = 16
NS = 16
NC = 2
CH = 88
NR = 4
NI = 6
PER = 12
NP = 10240
RPT = NP // NS


def _make_sc_spmm(nch):
  assert nch % PER == 0 and nch // PER >= 3
  mesh = plsc.VectorSubcoreMesh(core_axis_name="c", subcore_axis_name="s")

  @functools.partial(
      pl.kernel,
      out_type=(jax.ShapeDtypeStruct((NP, D), jnp.float32),
                jax.ShapeDtypeStruct((NP, D), jnp.float32)),
      mesh=mesh,
      scratch_types=[
          pltpu.VMEM((NI, CH), jnp.int32),
          pltpu.VMEM((NI, CH), jnp.int32),
          pltpu.VMEM((NI, CH), jnp.float32),
          pltpu.VMEM((NR, CH, D), jnp.float32),
          pltpu.VMEM_SHARED((NP, D), jnp.float32),
          pltpu.SemaphoreType.DMA((NI,)),
          pltpu.SemaphoreType.DMA((NR,)),
          pltpu.SemaphoreType.DMA((NR,)),
          pltpu.SemaphoreType.DMA,
      ],
      compiler_params=pltpu.CompilerParams(needs_layout_passes=False),
  )
  def kern(x_hbm, isrc, idst, iw, osrc, odst, ow, out_in, out_out,
           sidx, didx, wbuf, rows, acc, isem, gsem, ssem, zsem):
    cid = lax.axis_index("c")
    sid = lax.axis_index("s")
    zero16 = jnp.zeros((L,), jnp.float32)

    def run(src_h, dst_h, w_h):
      def zero_acc():
        @pl.loop(0, CH)
        def _(e):
          for c in range(D // L):
            rows[0, e, pl.ds(c * L, L)] = zero16

        nz = RPT // CH
        for kk in range(nz):
          pltpu.async_copy(
              rows.at[0], acc.at[pl.ds(sid * RPT + kk * CH, CH)], zsem)
        rem = RPT % CH
        if rem:
          pltpu.async_copy(
              rows.at[0].at[pl.ds(0, rem)],
              acc.at[pl.ds(sid * RPT + nz * CH, rem)], zsem)
        for kk in range(nz):
          pltpu.make_async_copy(
              rows.at[0], acc.at[pl.ds(sid * RPT + kk * CH, CH)], zsem).wait()
        if rem:
          pltpu.make_async_copy(
              rows.at[0].at[pl.ds(0, rem)],
              acc.at[pl.ds(sid * RPT + nz * CH, rem)], zsem).wait()
      def idx_fetch(j, bi):
        base = (sid * nch + j) * CH
        pltpu.async_copy(src_h.at[pl.ds(base, CH)], sidx.at[bi], isem.at[bi])
        pltpu.async_copy(dst_h.at[pl.ds(base, CH)], didx.at[bi], isem.at[bi])
        pltpu.async_copy(w_h.at[pl.ds(base, CH)], wbuf.at[bi], isem.at[bi])

      def idx_wait(bi):
        pltpu.make_async_copy(
            src_h.at[pl.ds(0, CH)], sidx.at[bi], isem.at[bi]).wait()
        pltpu.make_async_copy(
            dst_h.at[pl.ds(0, CH)], didx.at[bi], isem.at[bi]).wait()
        pltpu.make_async_copy(
            w_h.at[pl.ds(0, CH)], wbuf.at[bi], isem.at[bi]).wait()

      def gather_start(bi, br):
        pltpu.async_copy(x_hbm.at[sidx.at[bi]], rows.at[br], gsem.at[br])

      def wait_gather(bi, br):
        pltpu.make_async_copy(
            x_hbm.at[sidx.at[bi]], rows.at[br], gsem.at[br]).wait()

      def scatter_start(bi, br):
        pltpu.async_copy(rows.at[br], acc.at[didx.at[bi]], ssem.at[br],
                         add=True)

      def wait_scatter(bi, br):
        pltpu.make_async_copy(
            rows.at[br], acc.at[didx.at[bi]], ssem.at[br]).wait()

      def multiply(bi, br):
        @pl.loop(0, CH, unroll=2)
        def _(e):
          wsplat = plsc.load_gather(
              wbuf.at[bi], [jnp.zeros((L,), jnp.int32) + e])
          for c in range(D // L):
            sl = rows[br, e, pl.ds(c * L, L)]
            rows[br, e, pl.ds(c * L, L)] = sl * wsplat

      def slot(j, kmod):
        @pl.when(j >= 2)
        def _():
          wait_scatter((kmod - 2) % NI, (kmod - 2) % NR)

        @pl.when(j + 4 < nch)
        def _():
          idx_fetch(j + 4, (kmod + 4) % NI)

        @pl.when(j + 2 < nch)
        def _():
          idx_wait((kmod + 2) % NI)
          gather_start((kmod + 2) % NI, (kmod + 2) % NR)

        wait_gather(kmod % NI, kmod % NR)
        multiply(kmod % NI, kmod % NR)
        scatter_start(kmod % NI, kmod % NR)

      for jj in range(4):
        idx_fetch(jj, jj)
      zero_acc()
      plsc.subcore_barrier()
      idx_wait(0)
      gather_start(0, 0)
      idx_wait(1)
      gather_start(1, 1)

      ngrp = nch // PER

      @pl.loop(0, ngrp)
      def _(g):
        j0 = g * PER
        for k in range(PER):
          slot(j0 + k, k)

      wait_scatter((PER - 2) % NI, (PER - 2) % NR)
      wait_scatter((PER - 1) % NI, (PER - 1) % NR)

    @pl.when(cid == 0)
    def _():
      run(isrc, idst, iw)

    @pl.when(cid == 1)
    def _():
      run(osrc, odst, ow)

    plsc.subcore_barrier()
    sl = pl.ds(sid * RPT, RPT)

    @pl.when(cid == 0)
    def _():
      pltpu.sync_copy(acc.at[sl], out_in.at[sl])

    @pl.when(cid == 1)
    def _():
      pltpu.sync_copy(acc.at[sl], out_out.at[sl])

  return kern


_BR = 1000
_SCALE = 1.0 / math.sqrt(D)


def _gate_body(x_ref, a_ref, b_ref, w1_ref, b1_ref, w2_ref, b2_ref, cp_ref,
               o_ref):
  x = x_ref[...]
  a = a_ref[...]
  b = b_ref[...]
  dn = (((1,), (1,)), ((), ()))
  t1 = jax.nn.relu(
      lax.dot_general(x * a, w1_ref[...], dn,
                      preferred_element_type=jnp.float32) + b1_ref[...])
  s1 = jnp.sum(t1, axis=1, keepdims=True) * _SCALE
  t2 = jax.nn.relu(
      lax.dot_general(x * b, w2_ref[...], dn,
                      preferred_element_type=jnp.float32) + b2_ref[...])
  s2 = jnp.sum(t2, axis=1, keepdims=True) * _SCALE
  m = jnp.maximum(s1, s2)
  e1 = jnp.exp(s1 - m)
  e2 = jnp.exp(s2 - m)
  inv = 1.0 / (e1 + e2)
  nb = a * (e1 * inv) + b * (e2 * inv)
  o_ref[...] = x * cp_ref[0] + nb * cp_ref[1] + cp_ref[2]


def _tc_gate(h, inb, outb, W1_w, b1, W2_w, b2, cp):
  grid = N // _BR
  blk = lambda: pl.BlockSpec((_BR, D), lambda i: (i, 0))
  full = lambda r, c: pl.BlockSpec((r, c), lambda i: (0, 0))
  return pl.pallas_call(
      _gate_body,
      grid=(grid,),
      in_specs=[
          blk(), blk(), blk(),
          full(D, D), full(1, D), full(D, D), full(1, D),
          pl.BlockSpec(memory_space=pltpu.SMEM),
      ],
      out_specs=blk(),
      out_shape=jax.ShapeDtypeStruct((N, D), jnp.float32),
  )(h, inb, outb, W1_w, b1, W2_w, b2, cp)


def kernel(x, in_edge_index, in_edge_weight, out_edge_index, out_edge_weight,
           W1_w, W1_b, W2_w, W2_b, conv_w, conv_b):
  E = in_edge_index.shape[1]
  epw = -(-E // NS)
  nch = -(-epw // CH)
  nch = max(-(-nch // PER), 3) * PER
  EP = NS * nch * CH
  pad = EP - E

  def prep(ei, ew):
    fill = jnp.arange(pad, dtype=jnp.int32) % N
    src = jnp.concatenate([ei[1], fill])
    dst = jnp.concatenate([ei[0], fill])
    w = jnp.concatenate([ew, jnp.zeros((pad,), ew.dtype)])
    return src, dst, w

  isrc, idst, iw = prep(in_edge_index, in_edge_weight)
  osrc, odst, ow = prep(out_edge_index, out_edge_weight)

  spmm = _make_sc_spmm(nch)
  b1 = W1_b.reshape(1, D)
  b2 = W2_b.reshape(1, D)
  cp = jnp.concatenate([conv_w, conv_b.reshape(1)])

  h = x
  for _ in range(LAYERS):
    inb, outb = spmm(h, isrc, idst, iw, osrc, odst, ow)
    h = _tc_gate(h, inb, outb, W1_w, b1, W2_w, b2, cp)
  return h
